# baseline (speedup 1.0000x reference)
import functools

import jax
import jax.numpy as jnp
from jax import lax
from jax.experimental import pallas as pl
from jax.experimental.pallas import tpu as pltpu
from jax.experimental.pallas import tpu_sc as plsc

B = 16384
D = 128
NC = 2
NS = 16
L = 16
NW = NC * NS
BPW = B // NW
C = 128
NCHUNK = BPW // C
DU = 8


def _sc_body(hidx, ridx, tidx, ent, rel, w16, b16, conf_out, part_out,
             idxh_v, idxr_v, idxt_v, h0, r0, t0, h1, r1, t1, conf_v,
             w_v, b_v, part_v, p_tmp, sem0, sem1):
    cid = lax.axis_index("c")
    sid = lax.axis_index("s")
    wid = sid * NC + cid
    base = wid * BPW

    pltpu.sync_copy(hidx.at[pl.ds(base, BPW)], idxh_v)
    pltpu.sync_copy(ridx.at[pl.ds(base, BPW)], idxr_v)
    pltpu.sync_copy(tidx.at[pl.ds(base, BPW)], idxt_v)
    pltpu.sync_copy(w16, w_v)
    pltpu.sync_copy(b16, b_v)
    w = w_v[...]
    b = b_v[...]
    lane = lax.iota(jnp.int32, L)
    zero = jnp.zeros((L,), jnp.float32)
    bufs = ((h0, r0, t0, sem0), (h1, r1, t1, sem1))

    def fire(c):
        hb, rb, tb, sem = bufs[c % 2]
        return (
            pltpu.async_copy(ent.at[idxh_v.at[pl.ds(c * C, C)]], hb, sem),
            pltpu.async_copy(rel.at[idxr_v.at[pl.ds(c * C, C)]], rb, sem),
            pltpu.async_copy(ent.at[idxt_v.at[pl.ds(c * C, C)]], tb, sem),
        )

    sq4 = (zero, zero, zero, zero)
    pend = fire(0)
    for c in range(NCHUNK):
        for cp in pend:
            cp.wait()
        if c + 1 < NCHUNK:
            pend = fire(c + 1)
        hb, rb, tb, _ = bufs[c % 2]

        def gbody(g, sq4, hb=hb, rb=rb, tb=tb, c=c):
            @plsc.parallel_loop(0, L, carry=(*sq4, zero), unroll=2)
            def jloop(j, carry, hb=hb, rb=rb, tb=tb, g=g):
                s0, s1, s2, s3, pv = carry
                row = g * L + j
                hs = [hb[row, pl.ds(L * k, L)] for k in range(8)]
                ts = [tb[row, pl.ds(L * k, L)] for k in range(8)]
                rs = [rb[row, pl.ds(L * k, L)] for k in range(8)]
                ms = [(hs[k] * ts[k]) * rs[k] for k in range(8)]
                p_sum = jnp.sum(((ms[0] + ms[1]) + (ms[2] + ms[3])) +
                                ((ms[4] + ms[5]) + (ms[6] + ms[7])))
                pv = jnp.where(lane == j, p_sum, pv)
                accs = [s0, s1, s2, s3]
                vs = hs + ts + rs
                for k in range(24):
                    accs[k % 4] = accs[k % 4] + vs[k] * vs[k]
                return (*accs, pv)

            s0, s1, s2, s3, pv = jloop
            z = pv * w + b
            conf_v[pl.ds(c * C + g * L, L)] = 1.0 / (1.0 + jnp.exp(-z))
            return (s0, s1, s2, s3)

        sq4 = lax.fori_loop(0, C // L, gbody, sq4)

    part_v[...] = ((sq4[0] + sq4[1]) + (sq4[2] + sq4[3]))
    pltpu.sync_copy(conf_v, conf_out.at[pl.ds(base, BPW)])
    pltpu.sync_copy(part_v, part_out.at[wid])


_sc_call = functools.partial(
    pl.kernel,
    out_type=[
        jax.ShapeDtypeStruct((B,), jnp.float32),
        jax.ShapeDtypeStruct((NW, L), jnp.float32),
    ],
    mesh=plsc.VectorSubcoreMesh(core_axis_name="c", subcore_axis_name="s"),
    compiler_params=pltpu.CompilerParams(needs_layout_passes=False),
    scratch_types=[
        pltpu.VMEM((BPW,), jnp.int32),
        pltpu.VMEM((BPW,), jnp.int32),
        pltpu.VMEM((BPW,), jnp.int32),
        pltpu.VMEM((C, D), jnp.float32),
        pltpu.VMEM((C, D), jnp.float32),
        pltpu.VMEM((C, D), jnp.float32),
        pltpu.VMEM((C, D), jnp.float32),
        pltpu.VMEM((C, D), jnp.float32),
        pltpu.VMEM((C, D), jnp.float32),
        pltpu.VMEM((BPW,), jnp.float32),
        pltpu.VMEM((L,), jnp.float32),
        pltpu.VMEM((L,), jnp.float32),
        pltpu.VMEM((L,), jnp.float32),
        pltpu.VMEM((L, L), jnp.float32),
        pltpu.SemaphoreType.DMA,
        pltpu.SemaphoreType.DMA,
    ],
)(_sc_body)


def _finish_body(p_ref, o_ref):
    o_ref[0, 0] = jnp.sum(p_ref[...]) * (1.0 / (float(B) * float(B) * float(D)))


_finish = pl.pallas_call(
    _finish_body,
    out_shape=jax.ShapeDtypeStruct((1, 1), jnp.float32),
    out_specs=pl.BlockSpec(memory_space=pltpu.SMEM),
)


def kernel(x, entityEmbed, relationEmbed, lin_w, lin_b):
    x = x.astype(jnp.int32)
    hidx = x[:, 0]
    ridx = x[:, 1]
    tidx = x[:, 2]
    w16 = jnp.full((L,), lin_w[0, 0], jnp.float32)
    b16 = jnp.full((L,), lin_b[0], jnp.float32)
    conf, part = _sc_call(hidx, ridx, tidx, entityEmbed, relationEmbed,
                          w16, b16)
    r_score = _finish(part)[0, 0]
    return conf, r_score
